# baseline (speedup 1.0000x reference)
import jax
import jax.numpy as jnp
from jax import lax
from jax.experimental import pallas as pl
from jax.experimental.pallas import tpu as pltpu
from jax.experimental.pallas import tpu_sc as plsc

LANES = 16
NW = 32
SHIFT = 13
SROWS = 1 << SHIFT
CAP = 64


def _sc_pool(ids_m, tok_emb):
    B, TPAD = ids_m.shape
    V, D = tok_emb.shape
    BPW = B // NW
    NCH = TPAD // LANES
    ND = D // LANES
    NPASS = -(-V // SROWS)
    LAST = V - SROWS

    def body(ids_hbm, table_hbm, out_hbm,
             ids_v, bkt_v, rows_v, acc_v, cnt_v, counts_s, stab, sem):
        sid = lax.axis_index("s")
        wid = sid * 2 + lax.axis_index("c")
        base = wid * BPW
        pltpu.sync_copy(ids_hbm.at[pl.ds(base, BPW)], ids_v)

        dummy = jnp.full((LANES,), SROWS, dtype=jnp.int32)
        zero16 = jnp.zeros((LANES,), jnp.float32)
        ones = jnp.ones((LANES,), jnp.int32)
        iota = lax.iota(jnp.int32, LANES)
        gdn = lax.GatherDimensionNumbers(
            offset_dims=(), collapsed_slice_dims=(0,), start_index_map=(0,))

        def fill(i, carry):
            bkt_v[pl.ds(i * LANES, LANES)] = dummy
            return carry
        lax.fori_loop(0, (BPW * NPASS * CAP) // LANES, fill, 0)

        def clr(r, carry):
            for k in range(ND):
                acc_v[r, pl.ds(k * LANES, LANES)] = zero16
            return carry
        lax.fori_loop(0, BPW, clr, 0)

        for k in range(ND):
            rows_v[0, pl.ds(k * LANES, LANES)] = zero16

        @pl.when(sid == 0)
        def _():
            pltpu.sync_copy(rows_v.at[pl.ds(0, 1)], stab.at[pl.ds(SROWS, 1)])

        def route_row(r, carry):
            rbase = jnp.broadcast_to(r * (NPASS * CAP), (LANES,))
            cnt_v[pl.ds(0, LANES)] = jnp.zeros((LANES,), jnp.int32)

            def route_chunk(j, carry2):
                ids = ids_v[r, pl.ds(j * LANES, LANES)]
                am = ids >= 0
                s = jnp.where(am, jnp.right_shift(ids, SHIFT), NPASS)
                rel = ids - jnp.minimum(jnp.left_shift(s, SHIFT), LAST)
                rank, _last = plsc.scan_count(s, mask=am)
                cnt = cnt_v[pl.ds(0, LANES)]
                c_of_s = lax.gather(
                    cnt, s.reshape(LANES, 1), gdn, (1,),
                    mode=lax.GatherScatterMode.PROMISE_IN_BOUNDS)
                sbase = rbase + s * CAP
                pos = sbase + c_of_s + rank - 1
                pos = jnp.minimum(pos, sbase + (CAP - 1))
                plsc.store_scatter(bkt_v, [pos], rel, mask=am)
                plsc.addupdate_scatter(cnt_v, [s], ones, mask=am)
                return carry2

            lax.fori_loop(0, NCH, route_chunk, 0)
            cnt = cnt_v[pl.ds(0, LANES)]
            for p in range(NPASS):
                counts_s[r * NPASS + p] = cnt[p]
            return carry

        lax.fori_loop(0, BPW, route_row, 0)

        SLICE = SROWS // LANES
        def do_pass(p, carry):
            start = jnp.minimum(p * SROWS, LAST)
            plsc.subcore_barrier()
            pltpu.sync_copy(
                table_hbm.at[pl.ds(start + sid * SLICE, SLICE)],
                stab.at[pl.ds(sid * SLICE, SLICE)])
            plsc.subcore_barrier()

            def gath_row(r, carry2):
                k = counts_s[r * NPASS + p]
                nck = jnp.right_shift(k + (LANES - 1), 4)
                bbase = r * (NPASS * CAP) + p * CAP

                def chunk(c, acc):
                    pltpu.async_copy(
                        stab.at[bkt_v.at[pl.ds(bbase + c * LANES, LANES)]],
                        rows_v, sem)
                    pltpu.make_async_copy(
                        table_hbm.at[pl.ds(0, LANES)], rows_v, sem).wait()
                    for t in range(LANES):
                        acc = tuple(acc[kk] + rows_v[t, pl.ds(kk * LANES, LANES)]
                                    for kk in range(ND))
                    return acc

                acc0 = tuple(acc_v[r, pl.ds(kk * LANES, LANES)]
                             for kk in range(ND))
                acc = lax.fori_loop(0, nck, chunk, acc0)
                for kk in range(ND):
                    acc_v[r, pl.ds(kk * LANES, LANES)] = acc[kk]
                return carry2

            lax.fori_loop(0, BPW, gath_row, 0)
            return carry

        lax.fori_loop(0, NPASS, do_pass, 0)
        pltpu.sync_copy(acc_v, out_hbm.at[pl.ds(base, BPW)])

    mesh = plsc.VectorSubcoreMesh(core_axis_name="c", subcore_axis_name="s")
    f = pl.kernel(
        body,
        out_type=jax.ShapeDtypeStruct((B, D), jnp.float32),
        mesh=mesh,
        compiler_params=pltpu.CompilerParams(needs_layout_passes=False),
        scratch_types=[
            pltpu.VMEM((BPW, TPAD), jnp.int32),
            pltpu.VMEM((BPW * NPASS * CAP,), jnp.int32),
            pltpu.VMEM((LANES, D), jnp.float32),
            pltpu.VMEM((BPW, D), jnp.float32),
            pltpu.VMEM((LANES,), jnp.int32),
            pltpu.SMEM((BPW * NPASS,), jnp.int32),
            pltpu.VMEM_SHARED((SROWS + 1, D), jnp.float32),
            pltpu.SemaphoreType.DMA,
        ],
    )
    return f(ids_m, tok_emb)


def _tc_pos(mask, pos):
    B = mask.shape[0]
    D = pos.shape[1]

    def body(mask_ref, pos_ref, posacc_ref, cnt_ref):
        mf = mask_ref[...].astype(jnp.float32)
        cnt_ref[...] = jnp.sum(mf, axis=1, keepdims=True)
        posacc_ref[...] = lax.dot_general(
            mf, pos_ref[...], (((1,), (0,)), ((), ())),
            preferred_element_type=jnp.float32)

    return pl.pallas_call(
        body,
        out_shape=(jax.ShapeDtypeStruct((B, D), jnp.float32),
                   jax.ShapeDtypeStruct((B, 1), jnp.float32)),
    )(mask, pos)


def _tc_finish(acc, posacc, cnt, gamma, beta, W, bias):
    B, D = acc.shape
    NCLS = W.shape[0]

    def body(acc_ref, posacc_ref, cnt_ref, gamma_ref, beta_ref,
             w_ref, bias_ref, out_ref):
        cnt = cnt_ref[...]
        pooled = (acc_ref[...] + posacc_ref[...]) / jnp.maximum(cnt, 1.0)
        mu = jnp.mean(pooled, axis=1, keepdims=True)
        var = jnp.mean((pooled - mu) ** 2, axis=1, keepdims=True)
        h = (pooled - mu) * lax.rsqrt(var + 1e-5) * gamma_ref[...] + beta_ref[...]
        out_ref[...] = lax.dot_general(
            h, w_ref[...], (((1,), (1,)), ((), ())),
            preferred_element_type=jnp.float32) + bias_ref[...]

    return pl.pallas_call(
        body,
        out_shape=jax.ShapeDtypeStruct((B, NCLS), jnp.float32),
    )(acc, posacc, cnt, gamma, beta, W, bias)


def kernel(input_ids, attention_mask, tok_emb, pos_emb, gamma, beta, W, b):
    B, T = input_ids.shape
    D = tok_emb.shape[1]
    NCLS = W.shape[0]
    tpad = ((T + LANES - 1) // LANES) * LANES

    ids_m = jnp.where(attention_mask != 0, input_ids, -1)
    ids_m = jnp.pad(ids_m, ((0, 0), (0, tpad - T)), constant_values=-1)

    posacc, cnt = _tc_pos(attention_mask, pos_emb[:T])
    acc = _sc_pool(ids_m, tok_emb)
    logits = _tc_finish(
        acc, posacc, cnt,
        gamma.reshape(1, D), beta.reshape(1, D), W, b.reshape(1, NCLS))
    return logits


# speedup vs baseline: 1.2709x; 1.2709x over previous
import jax
import jax.numpy as jnp
from jax import lax
from jax.experimental import pallas as pl
from jax.experimental.pallas import tpu as pltpu
from jax.experimental.pallas import tpu_sc as plsc

LANES = 16
NW = 32
SHIFT = 13
SROWS = 1 << SHIFT
CAP = 48


def _sc_pool(ids_m, tok_emb):
    B, TPAD = ids_m.shape
    V, D = tok_emb.shape
    BPW = B // NW
    NCH = TPAD // LANES
    ND = D // LANES
    NPASS = -(-V // SROWS)
    LAST = V - SROWS

    def body(ids_hbm, table_hbm, out_hbm,
             ids_v, bkt_v, rows_v, acc_v, cnt_v, counts_s, stab,
             sem0, sem1, sem2):
        sems = (sem0, sem1, sem2)
        sid = lax.axis_index("s")
        wid = sid * 2 + lax.axis_index("c")
        base = wid * BPW
        pltpu.sync_copy(ids_hbm.at[pl.ds(base, BPW)], ids_v)

        dummy = jnp.full((LANES,), SROWS, dtype=jnp.int32)
        zero16 = jnp.zeros((LANES,), jnp.float32)
        ones = jnp.ones((LANES,), jnp.int32)
        iota = lax.iota(jnp.int32, LANES)
        gdn = lax.GatherDimensionNumbers(
            offset_dims=(), collapsed_slice_dims=(0,), start_index_map=(0,))

        def fill(i, carry):
            bkt_v[pl.ds(i * LANES, LANES)] = dummy
            return carry
        lax.fori_loop(0, (BPW * NPASS * CAP) // LANES, fill, 0)

        def clr(r, carry):
            for k in range(ND):
                acc_v[r, pl.ds(k * LANES, LANES)] = zero16
            return carry
        lax.fori_loop(0, BPW, clr, 0)

        for k in range(ND):
            rows_v[0, pl.ds(k * LANES, LANES)] = zero16

        @pl.when(sid == 0)
        def _():
            pltpu.sync_copy(rows_v.at[pl.ds(0, 1)], stab.at[pl.ds(SROWS, 1)])

        def route_row(r, carry):
            rbase = jnp.broadcast_to(r * (NPASS * CAP), (LANES,))
            cnt_v[pl.ds(0, LANES)] = jnp.zeros((LANES,), jnp.int32)

            def route_chunk(j, carry2):
                ids = ids_v[r, pl.ds(j * LANES, LANES)]
                am = ids >= 0
                s = jnp.where(am, jnp.right_shift(ids, SHIFT), NPASS)
                rel = ids - jnp.minimum(jnp.left_shift(s, SHIFT), LAST)
                rank, _last = plsc.scan_count(s, mask=am)
                cnt = cnt_v[pl.ds(0, LANES)]
                c_of_s = lax.gather(
                    cnt, s.reshape(LANES, 1), gdn, (1,),
                    mode=lax.GatherScatterMode.PROMISE_IN_BOUNDS)
                sbase = rbase + s * CAP
                pos = sbase + c_of_s + rank - 1
                pos = jnp.minimum(pos, sbase + (CAP - 1))
                plsc.store_scatter(bkt_v, [pos], rel, mask=am)
                plsc.addupdate_scatter(cnt_v, [s], ones, mask=am)
                return carry2

            lax.fori_loop(0, NCH, route_chunk, 0)
            cnt = cnt_v[pl.ds(0, LANES)]
            for p in range(NPASS):
                counts_s[r * NPASS + p] = cnt[p]
            return carry

        lax.fori_loop(0, BPW, route_row, 0)

        SLICE = SROWS // LANES
        def do_pass(p, carry):
            start = jnp.minimum(p * SROWS, LAST)
            plsc.subcore_barrier()
            pltpu.sync_copy(
                table_hbm.at[pl.ds(start + sid * SLICE, SLICE)],
                stab.at[pl.ds(sid * SLICE, SLICE)])
            plsc.subcore_barrier()

            def fire0(r, band):
                bb = r * (NPASS * CAP) + p * CAP
                pltpu.async_copy(
                    stab.at[bkt_v.at[pl.ds(bb, LANES)]],
                    rows_v.at[pl.ds(band * LANES, LANES)], sems[band])

            def drain(band):
                pltpu.make_async_copy(
                    table_hbm.at[pl.ds(0, LANES)],
                    rows_v.at[pl.ds(band * LANES, LANES)],
                    sems[band]).wait()

            def accum_band(band, acc):
                for t in range(LANES):
                    acc = tuple(
                        acc[kk] + rows_v[band * LANES + t,
                                         pl.ds(kk * LANES, LANES)]
                        for kk in range(ND))
                return acc

            def rest_chunks(r, acc):
                k = counts_s[r * NPASS + p]
                nck = jnp.right_shift(k + (LANES - 1), 4)
                bb = r * (NPASS * CAP) + p * CAP

                def chunk(c, a):
                    pltpu.async_copy(
                        stab.at[bkt_v.at[pl.ds(bb + c * LANES, LANES)]],
                        rows_v.at[pl.ds(2 * LANES, LANES)], sems[2])
                    drain(2)
                    return accum_band(2, a)

                return lax.fori_loop(1, nck, chunk, acc)

            fire0(0, 0)

            def pair(i, carry2):
                for half in (0, 1):
                    r = 2 * i + half
                    if half == 0:
                        fire0(r + 1, 1)
                    else:
                        @pl.when(i < BPW // 2 - 1)
                        def _():
                            fire0(r + 1, 0)
                    drain(half)
                    acc = tuple(acc_v[r, pl.ds(kk * LANES, LANES)]
                                for kk in range(ND))
                    acc = accum_band(half, acc)
                    acc = rest_chunks(r, acc)
                    for kk in range(ND):
                        acc_v[r, pl.ds(kk * LANES, LANES)] = acc[kk]
                return carry2

            lax.fori_loop(0, BPW // 2, pair, 0)
            return carry

        lax.fori_loop(0, NPASS, do_pass, 0)
        pltpu.sync_copy(acc_v, out_hbm.at[pl.ds(base, BPW)])

    mesh = plsc.VectorSubcoreMesh(core_axis_name="c", subcore_axis_name="s")
    f = pl.kernel(
        body,
        out_type=jax.ShapeDtypeStruct((B, D), jnp.float32),
        mesh=mesh,
        compiler_params=pltpu.CompilerParams(needs_layout_passes=False),
        scratch_types=[
            pltpu.VMEM((BPW, TPAD), jnp.int32),
            pltpu.VMEM((BPW * NPASS * CAP,), jnp.int32),
            pltpu.VMEM((3 * LANES, D), jnp.float32),
            pltpu.VMEM((BPW, D), jnp.float32),
            pltpu.VMEM((LANES,), jnp.int32),
            pltpu.SMEM((BPW * NPASS,), jnp.int32),
            pltpu.VMEM_SHARED((SROWS + 1, D), jnp.float32),
            pltpu.SemaphoreType.DMA,
            pltpu.SemaphoreType.DMA,
            pltpu.SemaphoreType.DMA,
        ],
    )
    return f(ids_m, tok_emb)


def _tc_pos(mask, pos):
    B = mask.shape[0]
    D = pos.shape[1]

    def body(mask_ref, pos_ref, posacc_ref, cnt_ref):
        mf = mask_ref[...].astype(jnp.float32)
        cnt_ref[...] = jnp.sum(mf, axis=1, keepdims=True)
        posacc_ref[...] = lax.dot_general(
            mf, pos_ref[...], (((1,), (0,)), ((), ())),
            preferred_element_type=jnp.float32)

    return pl.pallas_call(
        body,
        out_shape=(jax.ShapeDtypeStruct((B, D), jnp.float32),
                   jax.ShapeDtypeStruct((B, 1), jnp.float32)),
    )(mask, pos)


def _tc_finish(acc, posacc, cnt, gamma, beta, W, bias):
    B, D = acc.shape
    NCLS = W.shape[0]

    def body(acc_ref, posacc_ref, cnt_ref, gamma_ref, beta_ref,
             w_ref, bias_ref, out_ref):
        cnt = cnt_ref[...]
        pooled = (acc_ref[...] + posacc_ref[...]) / jnp.maximum(cnt, 1.0)
        mu = jnp.mean(pooled, axis=1, keepdims=True)
        var = jnp.mean((pooled - mu) ** 2, axis=1, keepdims=True)
        h = (pooled - mu) * lax.rsqrt(var + 1e-5) * gamma_ref[...] + beta_ref[...]
        out_ref[...] = lax.dot_general(
            h, w_ref[...], (((1,), (1,)), ((), ())),
            preferred_element_type=jnp.float32) + bias_ref[...]

    return pl.pallas_call(
        body,
        out_shape=jax.ShapeDtypeStruct((B, NCLS), jnp.float32),
    )(acc, posacc, cnt, gamma, beta, W, bias)


def kernel(input_ids, attention_mask, tok_emb, pos_emb, gamma, beta, W, b):
    B, T = input_ids.shape
    D = tok_emb.shape[1]
    NCLS = W.shape[0]
    tpad = ((T + LANES - 1) // LANES) * LANES

    ids_m = jnp.where(attention_mask != 0, input_ids, -1)
    ids_m = jnp.pad(ids_m, ((0, 0), (0, tpad - T)), constant_values=-1)

    posacc, cnt = _tc_pos(attention_mask, pos_emb[:T])
    acc = _sc_pool(ids_m, tok_emb)
    logits = _tc_finish(
        acc, posacc, cnt,
        gamma.reshape(1, D), beta.reshape(1, D), W, b.reshape(1, NCLS))
    return logits
